# baseline (speedup 1.0000x reference)
import functools

import jax
import jax.numpy as jnp
from jax import lax
from jax.experimental import pallas as pl
from jax.experimental.pallas import tpu as pltpu
from jax.experimental.pallas import tpu_sc as plsc

N = 10000
E = 320000
D = 128
HD = D // 2
NC = 2
NS = 16
NW = NC * NS
EPW = E // NW
C = 80
KC = EPW // C


def _pack_bf16(h):
    lo = lax.bitcast_convert_type(h[:, :HD].astype(jnp.bfloat16),
                                  jnp.uint16).astype(jnp.uint32)
    hi = lax.bitcast_convert_type(h[:, HD:].astype(jnp.bfloat16),
                                  jnp.uint16).astype(jnp.uint32)
    return lax.bitcast_convert_type(lo | (hi << 16), jnp.int32)


def _mm_body(a_ref, w_ref, o_ref):
    o_ref[...] = jnp.dot(a_ref[...], w_ref[...],
                         preferred_element_type=jnp.float32)


def _mm(a, w, blk):
    r = a.shape[0]
    return pl.pallas_call(
        _mm_body,
        grid=(r // blk,),
        in_specs=[pl.BlockSpec((blk, D), lambda i: (i, 0)),
                  pl.BlockSpec((D, D), lambda i: (0, 0))],
        out_specs=pl.BlockSpec((blk, D), lambda i: (i, 0)),
        out_shape=jax.ShapeDtypeStruct((r, D), jnp.float32),
    )(a, w)


def _mmh_body(a_ref, w_ref, o_ref):
    h = jnp.dot(a_ref[...], w_ref[...], preferred_element_type=jnp.float32)
    o_ref[...] = _pack_bf16(h)


def _mmh(a, w, blk):
    r = a.shape[0]
    return pl.pallas_call(
        _mmh_body,
        grid=(r // blk,),
        in_specs=[pl.BlockSpec((blk, D), lambda i: (i, 0)),
                  pl.BlockSpec((D, D), lambda i: (0, 0))],
        out_specs=pl.BlockSpec((blk, HD), lambda i: (i, 0)),
        out_shape=jax.ShapeDtypeStruct((r, HD), jnp.int32),
    )(a, w)


def _h2_mm_body(ea_ref, m_ref, w_ref, o_ref):
    pv = ea_ref[...] + m_ref[...]
    h = jnp.dot(pv, w_ref[...], preferred_element_type=jnp.float32)
    o_ref[...] = _pack_bf16(h)


def _h2_mm(ea, m, w, blk):
    return pl.pallas_call(
        _h2_mm_body,
        grid=(ea.shape[0] // blk,),
        in_specs=[pl.BlockSpec((blk, D), lambda i: (i, 0)),
                  pl.BlockSpec((blk, D), lambda i: (i, 0)),
                  pl.BlockSpec((D, D), lambda i: (0, 0))],
        out_specs=pl.BlockSpec((blk, HD), lambda i: (i, 0)),
        out_shape=jax.ShapeDtypeStruct((ea.shape[0], HD), jnp.int32),
    )(ea, m, w)


def _upd_body(x_ref, agg_ref, bgg_ref, wu_ref, b_ref, wm_ref, xo_ref, yo_ref):
    agg = (agg_ref[0] + agg_ref[1]) + (bgg_ref[0] + bgg_ref[1])
    xn = x_ref[...] + jnp.maximum(
        jnp.dot(agg, wu_ref[...], preferred_element_type=jnp.float32)
        + b_ref[...], 0.0)
    xo_ref[...] = xn
    yo_ref[...] = jnp.dot(xn, wm_ref[...], preferred_element_type=jnp.float32)


def _update(x, agga, aggb, wu, b2, wm, blk=2000):
    return pl.pallas_call(
        _upd_body,
        grid=(N // blk,),
        in_specs=[pl.BlockSpec((blk, D), lambda i: (i, 0)),
                  pl.BlockSpec((2, blk, D), lambda i: (0, i, 0)),
                  pl.BlockSpec((2, blk, D), lambda i: (0, i, 0)),
                  pl.BlockSpec((D, D), lambda i: (0, 0)),
                  pl.BlockSpec((1, D), lambda i: (0, 0)),
                  pl.BlockSpec((D, D), lambda i: (0, 0))],
        out_specs=[pl.BlockSpec((blk, D), lambda i: (i, 0)),
                   pl.BlockSpec((blk, D), lambda i: (i, 0))],
        out_shape=[jax.ShapeDtypeStruct((N, D), jnp.float32),
                   jax.ShapeDtypeStruct((N, D), jnp.float32)],
    )(x, agga, aggb, wu, b2, wm)


def _sc_body(with_m, EW, KCk, *refs):
    if with_m:
        (y_hbm, h_hbm, src_hbm, dst_hbm, z_hbm,
         agg_hbm, m_hbm,
         srci, dsti, gv, hv, aggsh, si, sg, sh, sa, sm) = refs
    else:
        (y_hbm, h_hbm, src_hbm, dst_hbm, z_hbm,
         agg_hbm,
         srci, dsti, gv, hv, aggsh, si, sg, sh, sa, sm) = refs
        m_hbm = None

    c = lax.axis_index("c")
    s = lax.axis_index("s")
    w = c * NS + s

    @pl.when(s == 0)
    def _zero():
        pltpu.sync_copy(z_hbm, aggsh)
    plsc.subcore_barrier()

    def issue_idx(k):
        p = jnp.bitwise_and(k, 3)
        pltpu.async_copy(src_hbm.at[w, k], srci.at[p], si.at[p])
        pltpu.async_copy(dst_hbm.at[w, k], dsti.at[p], si.at[p])

    def issue_in(k):
        p = jnp.bitwise_and(k, 1)
        pi = jnp.bitwise_and(k, 3)
        base = w * EW + k * C
        pltpu.async_copy(y_hbm.at[srci.at[pi]], gv.at[p], sg.at[p])
        pltpu.async_copy(h_hbm.at[pl.ds(base, C), :], hv.at[p], sh.at[p])

    def _drain(sem, dummy_src, dst):
        pltpu.make_async_copy(dummy_src, dst, sem).wait()

    pltpu.sync_copy(src_hbm.at[w, 0], srci.at[0])
    pltpu.sync_copy(dst_hbm.at[w, 0], dsti.at[0])
    issue_in(0)
    issue_idx(1)

    def body(k, carry):
        p = jnp.bitwise_and(k, 1)
        q = jnp.bitwise_and(k + 1, 1)
        pi = jnp.bitwise_and(k, 3)
        qi = jnp.bitwise_and(k + 1, 3)

        @pl.when(k < KCk - 1)
        def _():
            _drain(si.at[qi], src_hbm.at[w, 0], srci.at[qi])
            _drain(si.at[qi], src_hbm.at[w, 0], dsti.at[qi])

            @pl.when(k > 0)
            def _():
                _drain(sa.at[q], z_hbm.at[pl.ds(0, C), :], gv.at[q])
                if with_m:
                    _drain(sm.at[q], z_hbm.at[pl.ds(0, C), :], gv.at[q])
            issue_in(k + 1)
        _drain(sg.at[p], z_hbm.at[pl.ds(0, C), :], gv.at[p])
        _drain(sh.at[p], h_hbm.at[pl.ds(0, C), :], hv.at[p])

        def row(r, cr):
            for j in range(HD // 16):
                sl = pl.ds(j * 16, 16)
                sh_ = pl.ds(HD + j * 16, 16)
                vi = hv[p, r, sl]
                flo = lax.bitcast_convert_type(vi << 16, jnp.float32)
                fhi = lax.bitcast_convert_type(
                    jnp.bitwise_and(vi, jnp.int32(-65536)), jnp.float32)
                gv[p, r, sl] = jnp.maximum(gv[p, r, sl] + flo, 0.0)
                gv[p, r, sh_] = jnp.maximum(gv[p, r, sh_] + fhi, 0.0)
            return cr
        lax.fori_loop(0, C, row, 0, unroll=2)

        pltpu.async_copy(gv.at[p], aggsh.at[dsti.at[pi]], sa.at[p], add=True)
        if with_m:
            pltpu.async_copy(gv.at[p], m_hbm.at[pl.ds(w * EW + k * C, C), :],
                             sm.at[p])

        @pl.when(k < KCk - 2)
        def _():
            issue_idx(k + 2)
        return carry

    lax.fori_loop(0, KCk, body, 0)
    for slot in (0, 1):
        _drain(sa.at[slot], z_hbm.at[pl.ds(0, C), :], gv.at[slot])
        if with_m:
            _drain(sm.at[slot], z_hbm.at[pl.ds(0, C), :], gv.at[slot])
    plsc.subcore_barrier()

    @pl.when(s < NS - 1)
    def _dump_main():
        pltpu.sync_copy(aggsh.at[pl.ds(s * 640, 640), :],
                        agg_hbm.at[c, pl.ds(s * 640, 640), :])

    @pl.when(s == NS - 1)
    def _dump_tail():
        pltpu.sync_copy(aggsh.at[pl.ds(9600, 400), :],
                        agg_hbm.at[c, pl.ds(9600, 400), :])


_MESH = plsc.VectorSubcoreMesh(core_axis_name="c", subcore_axis_name="s",
                               num_cores=NC, num_subcores=NS)

_SC_SCRATCH = (
    [pltpu.VMEM((4, C), jnp.int32)] * 2
    + [pltpu.VMEM((2, C, D), jnp.float32)]
    + [pltpu.VMEM((2, C, HD), jnp.int32)]
    + [pltpu.VMEM_SHARED((N, D), jnp.float32)]
    + [pltpu.SemaphoreType.DMA((4,))]
    + [pltpu.SemaphoreType.DMA((2,))] * 4
)

_SC_PARAMS = pltpu.CompilerParams(use_tc_tiling_on_sc=False,
                                  internal_scratch_in_bytes=128 * 1024)

def _make_sc(with_m, ec):
    ew = ec // NW
    kck = ew // C
    out = jax.ShapeDtypeStruct((NC, N, D), jnp.float32)
    if with_m:
        out = [out, jax.ShapeDtypeStruct((ec, D), jnp.float32)]
    return pl.kernel(
        functools.partial(_sc_body, with_m, ew, kck),
        out_type=out,
        mesh=_MESH,
        scratch_types=_SC_SCRATCH,
        compiler_params=_SC_PARAMS,
    )


EA = 102400
EB = E - EA

_sc_m_a = _make_sc(True, EA)
_sc_m_b = _make_sc(True, EB)
_sc_nom_a = _make_sc(False, EA)
_sc_nom_b = _make_sc(False, EB)


def kernel(x, edge_index, edge_attr, W_msg, W_edge, W_upd, b_upd):
    src = edge_index[0].astype(jnp.int32)
    dst = edge_index[1].astype(jnp.int32)
    srca = src[:EA].reshape(NW, EA // NW // C, C)
    dsta = dst[:EA].reshape(NW, EA // NW // C, C)
    srcb = src[EA:].reshape(NW, EB // NW // C, C)
    dstb = dst[EA:].reshape(NW, EB // NW // C, C)
    eaa = edge_attr[:EA]
    eab = edge_attr[EA:]
    zeros = jnp.zeros((N, D), jnp.float32)
    b2 = b_upd.reshape(1, D)

    y1 = _mm(x, W_msg, 2000)
    h1a = _mmh(eaa, W_edge, 3200)
    agg1a, m1a = _sc_m_a(y1, h1a, srca, dsta, zeros)
    h1b = _mmh(eab, W_edge, 3200)
    agg1b, m1b = _sc_m_b(y1, h1b, srcb, dstb, zeros)
    h2a = _h2_mm(eaa, m1a, W_edge, 3200)
    x1, y2 = _update(x, agg1a, agg1b, W_upd, b2, W_msg)

    agg2a = _sc_nom_a(y2, h2a, srca, dsta, zeros)
    h2b = _h2_mm(eab, m1b, W_edge, 3200)
    agg2b = _sc_nom_b(y2, h2b, srcb, dstb, zeros)
    x2, _ = _update(x1, agg2a, agg2b, W_upd, b2, W_msg)
    return x2


# speedup vs baseline: 1.1160x; 1.1160x over previous
import functools

import jax
import jax.numpy as jnp
from jax import lax
from jax.experimental import pallas as pl
from jax.experimental.pallas import tpu as pltpu
from jax.experimental.pallas import tpu_sc as plsc

N = 10000
E = 320000
D = 128
HD = D // 2
NC = 2
NS = 16
NW = NC * NS
EPW = E // NW
C = 80
KC = EPW // C


def _pack_bf16(h):
    lo = lax.bitcast_convert_type(h[:, :HD].astype(jnp.bfloat16),
                                  jnp.uint16).astype(jnp.uint32)
    hi = lax.bitcast_convert_type(h[:, HD:].astype(jnp.bfloat16),
                                  jnp.uint16).astype(jnp.uint32)
    return lax.bitcast_convert_type(lo | (hi << 16), jnp.int32)


def _mm_body(a_ref, w_ref, o_ref):
    o_ref[...] = jnp.dot(a_ref[...], w_ref[...],
                         preferred_element_type=jnp.float32)


def _mm(a, w, blk):
    r = a.shape[0]
    return pl.pallas_call(
        _mm_body,
        grid=(r // blk,),
        in_specs=[pl.BlockSpec((blk, D), lambda i: (i, 0)),
                  pl.BlockSpec((D, D), lambda i: (0, 0))],
        out_specs=pl.BlockSpec((blk, D), lambda i: (i, 0)),
        out_shape=jax.ShapeDtypeStruct((r, D), jnp.float32),
    )(a, w)


def _mmh_body(a_ref, w_ref, o_ref):
    h = jnp.dot(a_ref[...], w_ref[...], preferred_element_type=jnp.float32)
    o_ref[...] = _pack_bf16(h)


def _mmh(a, w, blk):
    r = a.shape[0]
    return pl.pallas_call(
        _mmh_body,
        grid=(r // blk,),
        in_specs=[pl.BlockSpec((blk, D), lambda i: (i, 0)),
                  pl.BlockSpec((D, D), lambda i: (0, 0))],
        out_specs=pl.BlockSpec((blk, HD), lambda i: (i, 0)),
        out_shape=jax.ShapeDtypeStruct((r, HD), jnp.int32),
    )(a, w)


def _h2_mm_body(ea_ref, m_ref, w_ref, o_ref):
    pv = ea_ref[...] + m_ref[...]
    h = jnp.dot(pv, w_ref[...], preferred_element_type=jnp.float32)
    o_ref[...] = _pack_bf16(h)


def _h2_mm(ea, m, w, blk):
    return pl.pallas_call(
        _h2_mm_body,
        grid=(ea.shape[0] // blk,),
        in_specs=[pl.BlockSpec((blk, D), lambda i: (i, 0)),
                  pl.BlockSpec((blk, D), lambda i: (i, 0)),
                  pl.BlockSpec((D, D), lambda i: (0, 0))],
        out_specs=pl.BlockSpec((blk, HD), lambda i: (i, 0)),
        out_shape=jax.ShapeDtypeStruct((ea.shape[0], HD), jnp.int32),
    )(ea, m, w)


def _upd_body(x_ref, agg_ref, bgg_ref, wu_ref, b_ref, wm_ref, xo_ref, yo_ref):
    agg = (agg_ref[0] + agg_ref[1]) + (bgg_ref[0] + bgg_ref[1])
    xn = x_ref[...] + jnp.maximum(
        jnp.dot(agg, wu_ref[...], preferred_element_type=jnp.float32)
        + b_ref[...], 0.0)
    xo_ref[...] = xn
    yo_ref[...] = jnp.dot(xn, wm_ref[...], preferred_element_type=jnp.float32)


def _update(x, agga, aggb, wu, b2, wm, blk=2000):
    return pl.pallas_call(
        _upd_body,
        grid=(N // blk,),
        in_specs=[pl.BlockSpec((blk, D), lambda i: (i, 0)),
                  pl.BlockSpec((2, blk, D), lambda i: (0, i, 0)),
                  pl.BlockSpec((2, blk, D), lambda i: (0, i, 0)),
                  pl.BlockSpec((D, D), lambda i: (0, 0)),
                  pl.BlockSpec((1, D), lambda i: (0, 0)),
                  pl.BlockSpec((D, D), lambda i: (0, 0))],
        out_specs=[pl.BlockSpec((blk, D), lambda i: (i, 0)),
                   pl.BlockSpec((blk, D), lambda i: (i, 0))],
        out_shape=[jax.ShapeDtypeStruct((N, D), jnp.float32),
                   jax.ShapeDtypeStruct((N, D), jnp.float32)],
    )(x, agga, aggb, wu, b2, wm)


def _sc_body(with_m, EW, KCk, *refs):
    if with_m:
        (y_hbm, h_hbm, src_hbm, dst_hbm, z_hbm,
         agg_hbm, m_hbm,
         srci, dsti, gv, hv, aggsh, si, sg, sh, sa, sm) = refs
    else:
        (y_hbm, h_hbm, src_hbm, dst_hbm, z_hbm,
         agg_hbm,
         srci, dsti, gv, hv, aggsh, si, sg, sh, sa, sm) = refs
        m_hbm = None

    c = lax.axis_index("c")
    s = lax.axis_index("s")
    w = c * NS + s

    @pl.when(s == 0)
    def _zero():
        pltpu.sync_copy(z_hbm, aggsh)
    plsc.subcore_barrier()

    def issue_idx(k):
        p = jnp.bitwise_and(k, 3)
        pltpu.async_copy(src_hbm.at[w, k], srci.at[p], si.at[p])
        pltpu.async_copy(dst_hbm.at[w, k], dsti.at[p], si.at[p])

    def issue_in(k):
        p = jnp.bitwise_and(k, 1)
        pi = jnp.bitwise_and(k, 3)
        base = w * EW + k * C
        pltpu.async_copy(y_hbm.at[srci.at[pi]], gv.at[p], sg.at[p])
        pltpu.async_copy(h_hbm.at[pl.ds(base, C), :], hv.at[p], sh.at[p])

    def _drain(sem, dummy_src, dst):
        pltpu.make_async_copy(dummy_src, dst, sem).wait()

    pltpu.sync_copy(src_hbm.at[w, 0], srci.at[0])
    pltpu.sync_copy(dst_hbm.at[w, 0], dsti.at[0])
    issue_in(0)
    issue_idx(1)

    def body(k, carry):
        p = jnp.bitwise_and(k, 1)
        q = jnp.bitwise_and(k + 1, 1)
        pi = jnp.bitwise_and(k, 3)
        qi = jnp.bitwise_and(k + 1, 3)

        @pl.when(k < KCk - 1)
        def _():
            _drain(si.at[qi], src_hbm.at[w, 0], srci.at[qi])
            _drain(si.at[qi], src_hbm.at[w, 0], dsti.at[qi])

            @pl.when(k > 0)
            def _():
                _drain(sa.at[q], z_hbm.at[pl.ds(0, C), :], gv.at[q])
                if with_m:
                    _drain(sm.at[q], z_hbm.at[pl.ds(0, C), :], gv.at[q])
            issue_in(k + 1)
        _drain(sg.at[p], z_hbm.at[pl.ds(0, C), :], gv.at[p])
        _drain(sh.at[p], h_hbm.at[pl.ds(0, C), :], hv.at[p])

        def row(r, cr):
            for j in range(HD // 16):
                sl = pl.ds(j * 16, 16)
                sh_ = pl.ds(HD + j * 16, 16)
                vi = hv[p, r, sl]
                flo = lax.bitcast_convert_type(vi << 16, jnp.float32)
                fhi = lax.bitcast_convert_type(
                    jnp.bitwise_and(vi, jnp.int32(-65536)), jnp.float32)
                gv[p, r, sl] = jnp.maximum(gv[p, r, sl] + flo, 0.0)
                gv[p, r, sh_] = jnp.maximum(gv[p, r, sh_] + fhi, 0.0)
            return cr
        lax.fori_loop(0, C, row, 0, unroll=2)

        pltpu.async_copy(gv.at[p], aggsh.at[dsti.at[pi]], sa.at[p], add=True)
        if with_m:
            pltpu.async_copy(gv.at[p], m_hbm.at[pl.ds(w * EW + k * C, C), :],
                             sm.at[p])

        @pl.when(k < KCk - 2)
        def _():
            issue_idx(k + 2)
        return carry

    lax.fori_loop(0, KCk, body, 0)
    for slot in (0, 1):
        _drain(sa.at[slot], z_hbm.at[pl.ds(0, C), :], gv.at[slot])
        if with_m:
            _drain(sm.at[slot], z_hbm.at[pl.ds(0, C), :], gv.at[slot])
    plsc.subcore_barrier()

    @pl.when(s < NS - 1)
    def _dump_main():
        pltpu.sync_copy(aggsh.at[pl.ds(s * 640, 640), :],
                        agg_hbm.at[c, pl.ds(s * 640, 640), :])

    @pl.when(s == NS - 1)
    def _dump_tail():
        pltpu.sync_copy(aggsh.at[pl.ds(9600, 400), :],
                        agg_hbm.at[c, pl.ds(9600, 400), :])


_MESH = plsc.VectorSubcoreMesh(core_axis_name="c", subcore_axis_name="s",
                               num_cores=NC, num_subcores=NS)

_SC_SCRATCH = (
    [pltpu.VMEM((4, C), jnp.int32)] * 2
    + [pltpu.VMEM((2, C, D), jnp.float32)]
    + [pltpu.VMEM((2, C, HD), jnp.int32)]
    + [pltpu.VMEM_SHARED((N, D), jnp.float32)]
    + [pltpu.SemaphoreType.DMA((4,))]
    + [pltpu.SemaphoreType.DMA((2,))] * 4
)

_SC_PARAMS = pltpu.CompilerParams(use_tc_tiling_on_sc=False,
                                  internal_scratch_in_bytes=128 * 1024)

def _make_sc(with_m, ec):
    ew = ec // NW
    kck = ew // C
    out = jax.ShapeDtypeStruct((NC, N, D), jnp.float32)
    if with_m:
        out = [out, jax.ShapeDtypeStruct((ec, D), jnp.float32)]
    return pl.kernel(
        functools.partial(_sc_body, with_m, ew, kck),
        out_type=out,
        mesh=_MESH,
        scratch_types=_SC_SCRATCH,
        compiler_params=_SC_PARAMS,
    )


EA = 153600
EB = E - EA

_sc_m_a = _make_sc(True, EA)
_sc_m_b = _make_sc(True, EB)
_sc_nom_a = _make_sc(False, EA)
_sc_nom_b = _make_sc(False, EB)


def kernel(x, edge_index, edge_attr, W_msg, W_edge, W_upd, b_upd):
    src = edge_index[0].astype(jnp.int32)
    dst = edge_index[1].astype(jnp.int32)
    srca = src[:EA].reshape(NW, EA // NW // C, C)
    dsta = dst[:EA].reshape(NW, EA // NW // C, C)
    srcb = src[EA:].reshape(NW, EB // NW // C, C)
    dstb = dst[EA:].reshape(NW, EB // NW // C, C)
    eaa = edge_attr[:EA]
    eab = edge_attr[EA:]
    zeros = jnp.zeros((N, D), jnp.float32)
    b2 = b_upd.reshape(1, D)

    y1 = _mm(x, W_msg, 2000)
    h1a = _mmh(eaa, W_edge, 3200)
    agg1a, m1a = _sc_m_a(y1, h1a, srca, dsta, zeros)
    h1b = _mmh(eab, W_edge, 3200)
    agg1b, m1b = _sc_m_b(y1, h1b, srcb, dstb, zeros)
    h2a = _h2_mm(eaa, m1a, W_edge, 3200)
    x1, y2 = _update(x, agg1a, agg1b, W_upd, b2, W_msg)

    agg2a = _sc_nom_a(y2, h2a, srca, dsta, zeros)
    h2b = _h2_mm(eab, m1b, W_edge, 3200)
    agg2b = _sc_nom_b(y2, h2b, srcb, dstb, zeros)
    x2, _ = _update(x1, agg2a, agg2b, W_upd, b2, W_msg)
    return x2
